# baseline (speedup 1.0000x reference)
import functools

import jax
import jax.numpy as jnp
from jax import lax
from jax.experimental import pallas as pl
from jax.experimental.pallas import tpu as pltpu
from jax.experimental.pallas import tpu_sc as plsc

_L = 16


def _make_impl(B, NF, CARD, D, NC, NS):
    assert NF % NC == 0
    FPC = NF // NC
    BW = B // NS
    assert B % NS == 0 and BW % 128 == 0
    assert D % 8 == 0
    NT = D // 8
    C_MAIN = CARD // 128 * 128
    C_TAIL = CARD - C_MAIN
    CW = (C_MAIN // 128 + NS - 1) // NS * 128
    CW_LAST = C_MAIN - (NS - 1) * CW
    assert CW_LAST > 0 and CW_LAST % 128 == 0
    WL = CW_LAST + (128 if C_TAIL else 0)
    SW = C_MAIN + (128 if C_TAIL else 0)
    VW = max(CW, WL)
    NPH = 2 * NT

    mesh = plsc.VectorSubcoreMesh(core_axis_name="c", subcore_axis_name="s")

    @functools.partial(
        pl.kernel,
        mesh=mesh,
        out_type=jax.ShapeDtypeStruct((NF, D, B), jnp.float32),
        compiler_params=pltpu.CompilerParams(use_tc_tiling_on_sc=True),
        scratch_types=[
            pltpu.VMEM((8, VW), jnp.float32),
            pltpu.VMEM((BW,), jnp.int32),
            pltpu.VMEM((4 * BW,), jnp.int32),
            pltpu.VMEM((4 * BW,), jnp.float32),
            pltpu.VMEM((4 * BW,), jnp.float32),
            pltpu.VMEM((8, BW), jnp.float32),
            pltpu.VMEM((FPC * D * _L,), jnp.float32),
            pltpu.VMEM_SHARED((4 * SW,), jnp.float32),
            pltpu.VMEM_SHARED((4 * SW,), jnp.float32),
            pltpu.SemaphoreType.DMA,
            pltpu.SemaphoreType.DMA,
            pltpu.SemaphoreType.DMA,
            pltpu.SemaphoreType.DMA,
        ],
    )
    def k(xf_hbm, tt_hbm, tail_hbm, biasf_hbm, out_hbm,
          slab, vm_c, vm_gidx, flat0, flat1, vm_blk, vm_bias,
          sh0, sh1, sem_s, sem_p0, sem_p1, sem_g):
        cid = lax.axis_index("c")
        sid = lax.axis_index("s")
        c0 = sid * CW
        shs = (sh0, sh1)
        flats = (flat0, flat1)
        sem_ps = (sem_p0, sem_p1)

        pltpu.sync_copy(
            biasf_hbm.at[pl.ds(cid * FPC * D * _L, FPC * D * _L)], vm_bias)

        def stage_copies(f, t):
            main = (tt_hbm.at[f, pl.ds(8 * t, 8), pl.ds(c0, CW)],
                    slab.at[:, pl.ds(0, CW)])
            last = [(tt_hbm.at[f, pl.ds(8 * t, 8), pl.ds(c0, CW_LAST)],
                     slab.at[:, pl.ds(0, CW_LAST)])]
            if C_TAIL:
                last.append((tail_hbm.at[f, pl.ds(8 * t, 8)],
                             slab.at[:, pl.ds(CW_LAST, 128)]))
            return main, last

        def fire_stage(f, t):
            main, last = stage_copies(f, t)

            @pl.when(sid < NS - 1)
            def _():
                pltpu.async_copy(*main, sem_s)

            @pl.when(sid == NS - 1)
            def _():
                for src, dst in last:
                    pltpu.async_copy(src, dst, sem_s)

        def wait_stage(f, t):
            main, last = stage_copies(f, t)

            @pl.when(sid < NS - 1)
            def _():
                pltpu.make_async_copy(*main, sem_s).wait()

            @pl.when(sid == NS - 1)
            def _():
                for src, dst in last:
                    pltpu.make_async_copy(src, dst, sem_s).wait()

        def pub_copies(q):
            t, h = q // 2, q % 2
            buf = q % 2
            main, last = [], []
            for r in range(4):
                main.append((slab.at[4 * h + r, pl.ds(0, CW)],
                             shs[buf].at[pl.ds(r * SW + c0, CW)]))
                last.append((slab.at[4 * h + r, pl.ds(0, WL)],
                             shs[buf].at[pl.ds(r * SW + c0, WL)]))
            return main, last

        def fire_pub(q):
            main, last = pub_copies(q)
            sem = sem_ps[q % 2]

            @pl.when(sid < NS - 1)
            def _():
                for src, dst in main:
                    pltpu.async_copy(src, dst, sem)

            @pl.when(sid == NS - 1)
            def _():
                for src, dst in last:
                    pltpu.async_copy(src, dst, sem)

        def wait_pub(q):
            main, last = pub_copies(q)
            sem = sem_ps[q % 2]

            @pl.when(sid < NS - 1)
            def _():
                for src, dst in main:
                    pltpu.make_async_copy(src, dst, sem).wait()

            @pl.when(sid == NS - 1)
            def _():
                for src, dst in last:
                    pltpu.make_async_copy(src, dst, sem).wait()

        fire_stage(cid * FPC, 0)

        def feature_body(fi, _):
            f = cid * FPC + fi
            pltpu.sync_copy(xf_hbm.at[pl.ds(f * B + sid * BW, BW)], vm_c)

            def gidx_body(v, _):
                cvec = vm_c[pl.ds(v * _L, _L)]
                for r in range(4):
                    vm_gidx[pl.ds(r * BW + v * _L, _L)] = cvec + r * SW
                return _
            lax.fori_loop(0, BW // _L, gidx_body, None)

            wait_stage(f, 0)
            fire_pub(0)

            for q in range(NPH):
                t, h = q // 2, q % 2
                buf = q % 2
                wait_pub(q)
                if h == 1:
                    if t < NT - 1:
                        fire_stage(f, t + 1)
                    else:
                        @pl.when(fi < FPC - 1)
                        def _():
                            fire_stage(f + 1, 0)
                plsc.subcore_barrier()

                pltpu.async_copy(shs[buf].at[vm_gidx], flats[buf], sem_g)

                if q + 1 < NPH:
                    if (q + 1) % 2 == 0:
                        wait_stage(f, (q + 1) // 2)
                    fire_pub(q + 1)

                pltpu.make_async_copy(
                    shs[buf].at[vm_gidx], flats[buf], sem_g).wait()

                for r in range(4):
                    bvec = vm_bias[
                        pl.ds((fi * D + 8 * t + 4 * h + r) * _L, _L)]

                    def row_body(v, _, r=r, h=h, buf=buf, bvec=bvec):
                        sl = pl.ds(v * _L, _L)
                        vm_blk[4 * h + r, sl] = (
                            flats[buf][pl.ds(r * BW + v * _L, _L)] + bvec)
                        return _
                    lax.fori_loop(0, BW // _L, row_body, None)
                plsc.subcore_barrier()

                if h == 1:
                    pltpu.sync_copy(
                        vm_blk,
                        out_hbm.at[f, pl.ds(8 * t, 8), pl.ds(sid * BW, BW)])
            return _

        lax.fori_loop(0, FPC, feature_body, None)

    return k, C_MAIN, C_TAIL


def kernel(x, tables, bias):
    B, NF = x.shape
    NF2, CARD, D = tables.shape
    assert NF2 == NF
    info = plsc.get_sparse_core_info()
    NC, NS = info.num_cores, info.num_subcores

    impl, C_MAIN, C_TAIL = _make_impl(B, NF, CARD, D, NC, NS)
    xf = x.astype(jnp.int32).T.reshape(-1)
    tt = jnp.transpose(tables, (0, 2, 1))
    tail = jnp.transpose(tables[:, C_MAIN:, :], (0, 2, 1)) if C_TAIL \
        else jnp.zeros((NF, D, 0), tables.dtype)
    tail = jnp.pad(tail, ((0, 0), (0, 0), (0, 128 - tail.shape[2])))
    biasf = jnp.repeat(bias.reshape(-1)[:, None], _L, axis=1).reshape(-1)
    out3 = impl(xf, tt, tail, biasf)
    return jnp.transpose(out3, (2, 0, 1))


# speedup vs baseline: 1.0746x; 1.0746x over previous
import functools

import jax
import jax.numpy as jnp
from jax import lax
from jax.experimental import pallas as pl
from jax.experimental.pallas import tpu as pltpu
from jax.experimental.pallas import tpu_sc as plsc

_L = 16


def _make_impl(B, NF, CARD, D, NC, NS):
    assert NF % NC == 0
    FPC = NF // NC
    BW = B // NS
    assert B % NS == 0 and BW % 128 == 0
    assert D % 8 == 0
    NT = D // 8
    C_MAIN = CARD // 128 * 128
    C_TAIL = CARD - C_MAIN
    CW = (C_MAIN // 128 + NS - 1) // NS * 128
    CW_LAST = C_MAIN - (NS - 1) * CW
    assert CW_LAST > 0 and CW_LAST % 128 == 0
    WL = CW_LAST + (128 if C_TAIL else 0)
    SW = C_MAIN + (128 if C_TAIL else 0)
    VW = max(CW, WL)

    mesh = plsc.VectorSubcoreMesh(core_axis_name="c", subcore_axis_name="s")

    @functools.partial(
        pl.kernel,
        mesh=mesh,
        out_type=jax.ShapeDtypeStruct((NF, D, B), jnp.float32),
        compiler_params=pltpu.CompilerParams(use_tc_tiling_on_sc=True),
        scratch_types=[
            pltpu.VMEM((8, VW), jnp.float32),
            pltpu.VMEM((BW,), jnp.int32),
            pltpu.VMEM((4 * BW,), jnp.int32),
            pltpu.VMEM((4 * BW,), jnp.float32),
            pltpu.VMEM((8, BW), jnp.float32),
            pltpu.VMEM((FPC * D * _L,), jnp.float32),
            pltpu.VMEM_SHARED((4 * SW,), jnp.float32),
            pltpu.SemaphoreType.DMA,
            pltpu.SemaphoreType.DMA,
        ],
    )
    def k(xf_hbm, tt_hbm, tail_hbm, biasf_hbm, out_hbm,
          slab, vm_c, vm_gidx, vm_flat, vm_blk, vm_bias,
          sh_slab, sem_s, sem_g):
        cid = lax.axis_index("c")
        sid = lax.axis_index("s")
        c0 = sid * CW

        pltpu.sync_copy(
            biasf_hbm.at[pl.ds(cid * FPC * D * _L, FPC * D * _L)], vm_bias)

        def stage_copies(f, t):
            main = (tt_hbm.at[f, pl.ds(8 * t, 8), pl.ds(c0, CW)],
                    slab.at[:, pl.ds(0, CW)])
            last = [(tt_hbm.at[f, pl.ds(8 * t, 8), pl.ds(c0, CW_LAST)],
                     slab.at[:, pl.ds(0, CW_LAST)])]
            if C_TAIL:
                last.append((tail_hbm.at[f, pl.ds(8 * t, 8)],
                             slab.at[:, pl.ds(CW_LAST, 128)]))
            return main, last

        def fire_stage(f, t):
            main, last = stage_copies(f, t)

            @pl.when(sid < NS - 1)
            def _():
                pltpu.async_copy(*main, sem_s)

            @pl.when(sid == NS - 1)
            def _():
                for src, dst in last:
                    pltpu.async_copy(src, dst, sem_s)

        def wait_stage(f, t):
            main, last = stage_copies(f, t)

            @pl.when(sid < NS - 1)
            def _():
                pltpu.make_async_copy(*main, sem_s).wait()

            @pl.when(sid == NS - 1)
            def _():
                for src, dst in last:
                    pltpu.make_async_copy(src, dst, sem_s).wait()

        fire_stage(cid * FPC, 0)

        def feature_body(fi, _):
            f = cid * FPC + fi
            pltpu.sync_copy(xf_hbm.at[pl.ds(f * B + sid * BW, BW)], vm_c)

            def gidx_body(v, _):
                cvec = vm_c[pl.ds(v * _L, _L)]
                for r in range(4):
                    vm_gidx[pl.ds(r * BW + v * _L, _L)] = cvec + r * SW
                return _
            lax.fori_loop(0, BW // _L, gidx_body, None)

            for t in range(NT):
                for h in range(2):
                    if h == 0:
                        wait_stage(f, t)
                    for r in range(4):
                        @pl.when(sid < NS - 1)
                        def _(r=r, h=h):
                            pltpu.sync_copy(
                                slab.at[4 * h + r, pl.ds(0, CW)],
                                sh_slab.at[pl.ds(r * SW + c0, CW)])

                        @pl.when(sid == NS - 1)
                        def _(r=r, h=h):
                            pltpu.sync_copy(
                                slab.at[4 * h + r, pl.ds(0, WL)],
                                sh_slab.at[pl.ds(r * SW + c0, WL)])
                    if h == 1:
                        if t < NT - 1:
                            fire_stage(f, t + 1)
                        else:
                            @pl.when(fi < FPC - 1)
                            def _():
                                fire_stage(f + 1, 0)
                    plsc.subcore_barrier()

                    pltpu.async_copy(sh_slab.at[vm_gidx], vm_flat, sem_g)

                    pltpu.make_async_copy(
                        sh_slab.at[vm_gidx], vm_flat, sem_g).wait()

                    for r in range(4):
                        bvec = vm_bias[
                            pl.ds((fi * D + 8 * t + 4 * h + r) * _L, _L)]

                        def row_body(v, _, r=r, h=h, bvec=bvec):
                            sl = pl.ds(v * _L, _L)
                            vm_blk[4 * h + r, sl] = (
                                vm_flat[pl.ds(r * BW + v * _L, _L)] + bvec)
                            return _
                        lax.fori_loop(0, BW // _L, row_body, None)
                    plsc.subcore_barrier()

                pltpu.sync_copy(
                    vm_blk,
                    out_hbm.at[f, pl.ds(8 * t, 8), pl.ds(sid * BW, BW)])
            return _

        lax.fori_loop(0, FPC, feature_body, None)

    return k, C_MAIN, C_TAIL


def kernel(x, tables, bias):
    B, NF = x.shape
    NF2, CARD, D = tables.shape
    assert NF2 == NF
    info = plsc.get_sparse_core_info()
    NC, NS = info.num_cores, info.num_subcores

    impl, C_MAIN, C_TAIL = _make_impl(B, NF, CARD, D, NC, NS)
    xf = x.astype(jnp.int32).T.reshape(-1)
    tt = jnp.transpose(tables, (0, 2, 1))
    tail = jnp.transpose(tables[:, C_MAIN:, :], (0, 2, 1)) if C_TAIL \
        else jnp.zeros((NF, D, 0), tables.dtype)
    tail = jnp.pad(tail, ((0, 0), (0, 0), (0, 128 - tail.shape[2])))
    biasf = jnp.repeat(bias.reshape(-1)[:, None], _L, axis=1).reshape(-1)
    out3 = impl(xf, tt, tail, biasf)
    return jnp.transpose(out3, (2, 0, 1))
